# Initial kernel scaffold; baseline (speedup 1.0000x reference)
import jax
import jax.numpy as jnp
from jax.experimental import pallas as pl


def kernel(loc_data, conf_data, priors, has_lp_data, size_lp_data, offset_data, targets):
    raise NotImplementedError("write your pallas kernel here")


# speedup vs baseline: 56.8537x; 56.8537x over previous
import functools

import jax
import jax.numpy as jnp
from jax import lax
from jax.experimental import pallas as pl

B, P, O = 32, 32768, 8
NUM_CLASSES = 2
THRESHOLD = 0.5
NEGPOS_RATIO = 3
VAR0, VAR1 = 0.1, 0.2
PR, PC = 256, 128


def _smooth_l1(x):
    ax = jnp.abs(x)
    return jnp.where(ax < 1.0, 0.5 * x * x, ax - 0.5)


def _row_kernel(loc_ref, conf_ref, has_ref, size_ref, off_ref, pri_ref, tgt_ref,
                out_ref):
    f32 = jnp.float32
    loc = loc_ref[0]
    conf = conf_ref[0]
    hasd = has_ref[0]
    sized = size_ref[0]
    offd = off_ref[0]
    pri = pri_ref[...]

    pcx, pcy, pw, ph = pri[0], pri[1], pri[2], pri[3]
    px1 = pcx - pw / 2.0
    py1 = pcy - ph / 2.0
    px2 = pcx + pw / 2.0
    py2 = pcy + ph / 2.0
    area_b = (px2 - px1) * (py2 - py1)

    iota_r = lax.broadcasted_iota(jnp.int32, (PR, PC), 0)
    iota_c = lax.broadcasted_iota(jnp.int32, (PR, PC), 1)
    iota_flat = iota_r * PC + iota_c

    bto = jnp.full((PR, PC), -1.0, f32)
    bti = jnp.zeros((PR, PC), jnp.int32)
    bp_idx = []
    for j in range(O):
        ax1 = tgt_ref[0, j, 0]
        ay1 = tgt_ref[0, j, 1]
        ax2 = tgt_ref[0, j, 2]
        ay2 = tgt_ref[0, j, 3]
        area_a = (ax2 - ax1) * (ay2 - ay1)
        iw = jnp.clip(jnp.minimum(ax2, px2) - jnp.maximum(ax1, px1), 0.0, None)
        ih = jnp.clip(jnp.minimum(ay2, py2) - jnp.maximum(ay1, py1), 0.0, None)
        inter = iw * ih
        ratio = inter / (area_a + area_b - inter)
        m = jnp.max(ratio)
        bp_idx.append(jnp.min(jnp.where(ratio == m, iota_flat, jnp.int32(P))))
        upd = ratio > bto
        bto = jnp.where(upd, ratio, bto)
        bti = jnp.where(upd, j, bti)

    for j in range(O):
        msk = iota_flat == bp_idx[j]
        bto = jnp.where(msk, 2.0, bto)
        bti = jnp.where(msk, j, bti)

    def gather_col(c):
        v = jnp.full((PR, PC), tgt_ref[0, 0, c], f32)
        for j in range(1, O):
            v = jnp.where(bti == j, tgt_ref[0, j, c], v)
        return v

    mx1 = gather_col(0)
    my1 = gather_col(1)
    mx2 = gather_col(2)
    my2 = gather_col(3)
    mhl = gather_col(4)
    msz0 = gather_col(5)
    msz1 = gather_col(6)
    mof0 = gather_col(7)
    mof1 = gather_col(8)
    mlab = gather_col(9)

    pos = bto >= THRESHOLD
    posf = pos.astype(f32)
    conf_t = jnp.where(pos, mlab.astype(jnp.int32) + 1, 0)

    vpw = VAR0 * pw
    vph = VAR0 * ph
    lt0 = ((mx1 + mx2) / 2.0 - pcx) / vpw
    lt1 = ((my1 + my2) / 2.0 - pcy) / vph
    lt2 = jnp.log((mx2 - mx1) / pw) / VAR1
    lt3 = jnp.log((my2 - my1) / ph) / VAR1
    loss_l = jnp.sum((_smooth_l1(loc[0] - lt0) + _smooth_l1(loc[1] - lt1) +
                      _smooth_l1(loc[2] - lt2) + _smooth_l1(loc[3] - lt3)) * posf)

    hl = mhl.astype(jnp.int32).astype(f32)
    st0 = msz0 / pw
    st1 = msz1 / ph
    loss_sz = jnp.sum((_smooth_l1(sized[0] * hl - st0 * hl) +
                       _smooth_l1(sized[1] * hl - st1 * hl)) * posf)
    ot0 = (mof0 - pcx) / vpw
    ot1 = (mof1 - pcy) / vph
    loss_of = jnp.sum((_smooth_l1(offd[0] * hl - ot0 * hl) +
                       _smooth_l1(offd[1] * hl - ot1 * hl)) * posf)

    c0, c1 = conf[0], conf[1]
    cm = jnp.maximum(c0, c1)
    lse = cm + jnp.log(jnp.exp(c0 - cm) + jnp.exp(c1 - cm))
    gathered = jnp.where(conf_t >= 1, c1, c0)
    c_term = lse - gathered

    h0, h1 = hasd[0], hasd[1]
    hm = jnp.maximum(h0, h1)
    lse_h = hm + jnp.log(jnp.exp(h0 - hm) + jnp.exp(h1 - hm))
    g_h = jnp.where(mhl.astype(jnp.int32) >= 1, h1, h0)
    h_term = lse_h - g_h

    pos_c = jnp.sum(jnp.where(pos, c_term, 0.0))
    pos_h = jnp.sum(jnp.where(pos, h_term, 0.0))
    num_pos = jnp.sum(pos.astype(jnp.int32))

    lcm = jnp.where(pos, 0.0, c_term)
    key = jnp.where(pos, jnp.int32(-1),
                    lax.bitcast_convert_type(lcm, jnp.int32))
    keff = jnp.minimum(jnp.minimum(NEGPOS_RATIO * num_pos, P - 1), P - num_pos)

    def vbody(_, lh):
        lo, hi = lh
        mid = lo + (hi - lo) // 2
        cnt = jnp.sum(jnp.where(key >= mid, 1, 0))
        take = cnt >= keff
        return jnp.where(take, mid, lo), jnp.where(take, hi, mid)

    tau, _ = lax.fori_loop(0, 31, vbody, (jnp.int32(0), jnp.int32(2**31 - 1)))
    gt = key > tau
    cnt_gt = jnp.sum(jnp.where(gt, 1, 0))
    tie = key == tau
    tie_need = keff - cnt_gt

    def ibody(_, lh):
        lo, hi = lh
        mid = lo + (hi - lo) // 2
        cnt = jnp.sum(jnp.where(tie & (iota_flat < mid), 1, 0))
        take = cnt >= tie_need
        return jnp.where(take, lo, mid), jnp.where(take, mid, hi)

    _, cut = lax.fori_loop(0, 16, ibody, (jnp.int32(0), jnp.int32(P)))
    tie_sel = tie & (iota_flat < cut)

    tau_val = jnp.where(tie_need > 0,
                        lax.bitcast_convert_type(jnp.maximum(tau, 0), f32), 0.0)
    neg_c = jnp.sum(jnp.where(gt, lcm, 0.0)) + tie_need.astype(f32) * tau_val
    neg_h = (jnp.sum(jnp.where(gt, h_term, 0.0)) +
             jnp.sum(jnp.where(tie_sel, h_term, 0.0)))

    iota_o = lax.broadcasted_iota(jnp.int32, (PC,), 0)
    vals = [loss_l, pos_c + neg_c, loss_sz, loss_of, pos_h + neg_h,
            num_pos.astype(f32)]
    acc = jnp.zeros((PC,), f32)
    for i, v in enumerate(vals):
        acc = acc + jnp.where(iota_o == i, v, 0.0)
    out_ref[0, 0, :] = acc


def kernel(loc_data, conf_data, priors, has_lp_data, size_lp_data, offset_data,
           targets):
    f32 = jnp.float32
    locT = loc_data.transpose(0, 2, 1).reshape(B, 4, PR, PC)
    confT = conf_data.transpose(0, 2, 1).reshape(B, 2, PR, PC)
    hasT = has_lp_data.transpose(0, 2, 1).reshape(B, 2, PR, PC)
    sizeT = size_lp_data.transpose(0, 2, 1).reshape(B, 2, PR, PC)
    offT = offset_data.transpose(0, 2, 1).reshape(B, 2, PR, PC)
    priT = priors.transpose(1, 0).reshape(4, PR, PC)
    tgt = targets.reshape(B, O, 10)

    grid = (B,)
    partials = pl.pallas_call(
        _row_kernel,
        grid=grid,
        in_specs=[
            pl.BlockSpec((1, 4, PR, PC), lambda i: (i, 0, 0, 0)),
            pl.BlockSpec((1, 2, PR, PC), lambda i: (i, 0, 0, 0)),
            pl.BlockSpec((1, 2, PR, PC), lambda i: (i, 0, 0, 0)),
            pl.BlockSpec((1, 2, PR, PC), lambda i: (i, 0, 0, 0)),
            pl.BlockSpec((1, 2, PR, PC), lambda i: (i, 0, 0, 0)),
            pl.BlockSpec((4, PR, PC), lambda i: (0, 0, 0)),
            pl.BlockSpec((1, O, 10), lambda i: (i, 0, 0)),
        ],
        out_specs=pl.BlockSpec((1, 1, PC), lambda i: (i, 0, 0)),
        out_shape=jax.ShapeDtypeStruct((B, 1, PC), f32),
    )(locT, confT, hasT, sizeT, offT, priT, tgt)

    part = partials[:, 0, :6]
    sums = jnp.sum(part, axis=0)
    n = sums[5]
    return (sums[0] / n, sums[1] / n, sums[2] / n, sums[3] / n, sums[4] / n)


# speedup vs baseline: 113.6761x; 1.9994x over previous
import functools

import jax
import jax.numpy as jnp
from jax import lax
from jax.experimental import pallas as pl
from jax.experimental.pallas import tpu as pltpu

B, P, O = 32, 32768, 8
NUM_CLASSES = 2
THRESHOLD = 0.5
NEGPOS_RATIO = 3
VAR0, VAR1 = 0.1, 0.2
PR, PC = 256, 128


def _smooth_l1(x):
    ax = jnp.abs(x)
    return jnp.where(ax < 1.0, 0.5 * x * x, ax - 0.5)


def _dense_kernel(loc_ref, conf_ref, has_ref, size_ref, off_ref, pri_ref,
                  tgt_ref, part_ref, key_ref, hterm_ref):
    f32 = jnp.float32
    loc = loc_ref[0]
    conf = conf_ref[0]
    hasd = has_ref[0]
    sized = size_ref[0]
    offd = off_ref[0]
    pri = pri_ref[...]

    pcx, pcy, pw, ph = pri[0], pri[1], pri[2], pri[3]
    px1 = pcx - pw / 2.0
    py1 = pcy - ph / 2.0
    px2 = pcx + pw / 2.0
    py2 = pcy + ph / 2.0
    area_b = (px2 - px1) * (py2 - py1)

    iota_r = lax.broadcasted_iota(jnp.int32, (PR, PC), 0)
    iota_c = lax.broadcasted_iota(jnp.int32, (PR, PC), 1)
    iota_flat = iota_r * PC + iota_c

    ts = [[tgt_ref[0, j, c] for c in range(10)] for j in range(O)]

    bto = jnp.full((PR, PC), -1.0, f32)
    bti = jnp.zeros((PR, PC), jnp.int32)
    bp_idx = []
    for j in range(O):
        ax1, ay1, ax2, ay2 = ts[j][0], ts[j][1], ts[j][2], ts[j][3]
        area_a = (ax2 - ax1) * (ay2 - ay1)
        iw = jnp.clip(jnp.minimum(ax2, px2) - jnp.maximum(ax1, px1), 0.0, None)
        ih = jnp.clip(jnp.minimum(ay2, py2) - jnp.maximum(ay1, py1), 0.0, None)
        inter = iw * ih
        ratio = inter / (area_a + area_b - inter)
        m = jnp.max(ratio)
        bp_idx.append(jnp.min(jnp.where(ratio == m, iota_flat, jnp.int32(P))))
        upd = ratio > bto
        bto = jnp.where(upd, ratio, bto)
        bti = jnp.where(upd, j, bti)

    forced = jnp.zeros((PR, PC), jnp.bool_)
    for j in range(O):
        msk = iota_flat == bp_idx[j]
        forced = forced | msk
        bti = jnp.where(msk, j, bti)

    pos = (bto >= THRESHOLD) | forced
    posf = pos.astype(f32)

    m0 = (bti & 1) == 1
    m1 = (bti & 2) == 2
    m2 = (bti & 4) == 4

    def gather(vals):
        a0 = jnp.where(m0, vals[1], vals[0])
        a1 = jnp.where(m0, vals[3], vals[2])
        a2 = jnp.where(m0, vals[5], vals[4])
        a3 = jnp.where(m0, vals[7], vals[6])
        b0 = jnp.where(m1, a1, a0)
        b1 = jnp.where(m1, a3, a2)
        return jnp.where(m2, b1, b0)

    acx = gather([(ts[j][0] + ts[j][2]) / 2.0 for j in range(O)])
    acy = gather([(ts[j][1] + ts[j][3]) / 2.0 for j in range(O)])
    law = gather([jnp.log(ts[j][2] - ts[j][0]) for j in range(O)])
    lah = gather([jnp.log(ts[j][3] - ts[j][1]) for j in range(O)])
    hl = gather([ts[j][4] for j in range(O)])
    sz0 = gather([ts[j][5] for j in range(O)])
    sz1 = gather([ts[j][6] for j in range(O)])
    of0 = gather([ts[j][7] for j in range(O)])
    of1 = gather([ts[j][8] for j in range(O)])

    vpw = VAR0 * pw
    vph = VAR0 * ph
    lt0 = (acx - pcx) / vpw
    lt1 = (acy - pcy) / vph
    lt2 = (law - jnp.log(pw)) / VAR1
    lt3 = (lah - jnp.log(ph)) / VAR1
    loss_l = jnp.sum((_smooth_l1(loc[0] - lt0) + _smooth_l1(loc[1] - lt1) +
                      _smooth_l1(loc[2] - lt2) + _smooth_l1(loc[3] - lt3)) * posf)

    loss_sz = jnp.sum((_smooth_l1((sized[0] - sz0 / pw) * hl) +
                       _smooth_l1((sized[1] - sz1 / ph) * hl)) * posf)
    loss_of = jnp.sum((_smooth_l1((offd[0] - (of0 - pcx) / vpw) * hl) +
                       _smooth_l1((offd[1] - (of1 - pcy) / vph) * hl)) * posf)

    c0, c1 = conf[0], conf[1]
    cm = jnp.maximum(c0, c1)
    lse = cm + jnp.log(jnp.exp(c0 - cm) + jnp.exp(c1 - cm))
    gathered = jnp.where(pos, c1, c0)
    c_term = lse - gathered

    h0, h1 = hasd[0], hasd[1]
    hm = jnp.maximum(h0, h1)
    lse_h = hm + jnp.log(jnp.exp(h0 - hm) + jnp.exp(h1 - hm))
    g_h = jnp.where(hl >= 0.5, h1, h0)
    h_term = lse_h - g_h

    pos_c = jnp.sum(jnp.where(pos, c_term, 0.0))
    pos_h = jnp.sum(jnp.where(pos, h_term, 0.0))
    num_pos = jnp.sum(posf)

    key = jnp.where(pos, jnp.int32(-1),
                    lax.bitcast_convert_type(c_term, jnp.int32))
    keff = jnp.minimum(jnp.minimum(NEGPOS_RATIO * num_pos, float(P - 1)),
                       float(P) - num_pos)

    key_ref[0] = key
    hterm_ref[0] = h_term

    iota_o = lax.broadcasted_iota(jnp.int32, (PC,), 0)
    vals = [loss_l, loss_sz, loss_of, pos_c, pos_h, num_pos, keff]
    acc = jnp.zeros((PC,), f32)
    for i, v in enumerate(vals):
        acc = acc + jnp.where(iota_o == i, v, 0.0)
    part_ref[0, 0, :] = acc


def _select_kernel(key_ref, hterm_ref, part_ref, out_ref):
    f32 = jnp.float32
    key = key_ref[...]
    hterm = hterm_ref[...]
    part = part_ref[...]
    keff = part[:, :, 6:7].astype(jnp.int32)

    iota_flat = (lax.broadcasted_iota(jnp.int32, (1, PR, PC), 1) * PC +
                 lax.broadcasted_iota(jnp.int32, (1, PR, PC), 2))

    def vbody(_, lh):
        lo, hi = lh
        mid = lo + (hi - lo) // 2
        cnt = jnp.sum(jnp.where(key >= mid, 1, 0), axis=(1, 2), keepdims=True)
        take = cnt >= keff
        return jnp.where(take, mid, lo), jnp.where(take, hi, mid)

    lo0 = jnp.zeros((B, 1, 1), jnp.int32)
    hi0 = jnp.full((B, 1, 1), 2**31 - 1, jnp.int32)
    tau, _ = lax.fori_loop(0, 31, vbody, (lo0, hi0))

    gt = key > tau
    cnt_gt = jnp.sum(jnp.where(gt, 1, 0), axis=(1, 2), keepdims=True)
    tie = key == tau
    tie_need = keff - cnt_gt

    def ibody(_, lh):
        lo, hi = lh
        mid = lo + (hi - lo) // 2
        cnt = jnp.sum(jnp.where(tie & (iota_flat < mid), 1, 0),
                      axis=(1, 2), keepdims=True)
        take = cnt >= tie_need
        return jnp.where(take, lo, mid), jnp.where(take, mid, hi)

    zi = jnp.zeros((B, 1, 1), jnp.int32)
    _, cut = lax.fori_loop(0, 16, ibody, (zi, jnp.full((B, 1, 1), P, jnp.int32)))
    tie_sel = tie & (iota_flat < cut)

    lcm = lax.bitcast_convert_type(jnp.maximum(key, 0), f32)
    tau_val = jnp.where(tie_need > 0,
                        lax.bitcast_convert_type(jnp.maximum(tau, 0), f32), 0.0)
    neg_c = (jnp.sum(jnp.where(gt, lcm, 0.0)) +
             jnp.sum(tie_need.astype(f32) * tau_val))
    neg_h = jnp.sum(jnp.where(gt | tie_sel, hterm, 0.0))

    sums = jnp.sum(part[:, 0, :], axis=0)
    n = sums[5]
    vals = [sums[0] / n, (sums[3] + neg_c) / n, sums[1] / n, sums[2] / n,
            (sums[4] + neg_h) / n]
    iota_o = lax.broadcasted_iota(jnp.int32, (PC,), 0)
    acc = jnp.zeros((PC,), f32)
    for i, v in enumerate(vals):
        acc = acc + jnp.where(iota_o == i, v, 0.0)
    out_ref[0, 0, :] = acc


def kernel(loc_data, conf_data, priors, has_lp_data, size_lp_data, offset_data,
           targets):
    f32 = jnp.float32
    locT = loc_data.transpose(0, 2, 1).reshape(B, 4, PR, PC)
    confT = conf_data.transpose(0, 2, 1).reshape(B, 2, PR, PC)
    hasT = has_lp_data.transpose(0, 2, 1).reshape(B, 2, PR, PC)
    sizeT = size_lp_data.transpose(0, 2, 1).reshape(B, 2, PR, PC)
    offT = offset_data.transpose(0, 2, 1).reshape(B, 2, PR, PC)
    priT = priors.transpose(1, 0).reshape(4, PR, PC)
    tgt = targets.reshape(B, O, 10)

    part, key, hterm = pl.pallas_call(
        _dense_kernel,
        grid=(B,),
        in_specs=[
            pl.BlockSpec((1, 4, PR, PC), lambda i: (i, 0, 0, 0)),
            pl.BlockSpec((1, 2, PR, PC), lambda i: (i, 0, 0, 0)),
            pl.BlockSpec((1, 2, PR, PC), lambda i: (i, 0, 0, 0)),
            pl.BlockSpec((1, 2, PR, PC), lambda i: (i, 0, 0, 0)),
            pl.BlockSpec((1, 2, PR, PC), lambda i: (i, 0, 0, 0)),
            pl.BlockSpec((4, PR, PC), lambda i: (0, 0, 0)),
            pl.BlockSpec((1, O, 10), lambda i: (i, 0, 0),
                         memory_space=pltpu.SMEM),
        ],
        out_specs=[
            pl.BlockSpec((1, 1, PC), lambda i: (i, 0, 0)),
            pl.BlockSpec((1, PR, PC), lambda i: (i, 0, 0)),
            pl.BlockSpec((1, PR, PC), lambda i: (i, 0, 0)),
        ],
        out_shape=[
            jax.ShapeDtypeStruct((B, 1, PC), f32),
            jax.ShapeDtypeStruct((B, PR, PC), jnp.int32),
            jax.ShapeDtypeStruct((B, PR, PC), f32),
        ],
    )(locT, confT, hasT, sizeT, offT, priT, tgt)

    out = pl.pallas_call(
        _select_kernel,
        grid=(1,),
        in_specs=[
            pl.BlockSpec((B, PR, PC), lambda i: (0, 0, 0)),
            pl.BlockSpec((B, PR, PC), lambda i: (0, 0, 0)),
            pl.BlockSpec((B, 1, PC), lambda i: (0, 0, 0)),
        ],
        out_specs=pl.BlockSpec((1, 1, PC), lambda i: (0, 0, 0)),
        out_shape=jax.ShapeDtypeStruct((1, 1, PC), f32),
    )(key, hterm, part)

    return (out[0, 0, 0], out[0, 0, 1], out[0, 0, 2], out[0, 0, 3],
            out[0, 0, 4])
